# baseline (speedup 1.0000x reference)
import functools

import jax
import jax.numpy as jnp
from jax import lax
from jax.experimental import pallas as pl
from jax.experimental.pallas import tpu as pltpu
from jax.experimental.pallas import tpu_sc as plsc

_HW = 196
_K = 512
_CD = 32
_L = 12
_B = 2
_NW = 32
_CHUNK = _B * _L * _CD * _HW // _NW
_ROWS_PER_W = 24


def _tc_body(z_ref, codes_ref, idx_ref, g_ref):
    c = codes_ref[0]
    h = z_ref[0, 0]
    scores = jax.lax.dot_general(
        c, h, (((1,), (0,)), ((), ())),
        preferred_element_type=jnp.float32,
        precision=jax.lax.Precision.HIGHEST)
    cn = jnp.sum(c * c, axis=1, keepdims=True)
    d2 = cn - 2.0 * scores
    m = jnp.min(d2, axis=0, keepdims=True)
    kiota = jax.lax.broadcasted_iota(jnp.int32, d2.shape, 0)
    idx = jnp.min(jnp.where(d2 == m, kiota, _K), axis=0)
    idx_ref[0, 0, 0, :] = idx
    b = pl.program_id(0)
    l = pl.program_id(1)
    i = jax.lax.broadcasted_iota(jnp.int32, (_CD, _HW), 0)
    sect = (32 * l + i) // _ROWS_PER_W % 16
    lbase = (_ROWS_PER_W * sect) // 32
    g_ref[0, 0] = (l - lbase) * (_K * _CD) + idx[None, :] * _CD + i


def _sc_body(codes_hbm, g_hbm, out_hbm, cbuf, gbuf, obuf):
    wid = lax.axis_index("c") * 16 + lax.axis_index("s")
    sect = wid % 16
    lbase = (_ROWS_PER_W * sect) // 32
    pltpu.sync_copy(codes_hbm.at[pl.ds(lbase * _K * _CD, 2 * _K * _CD)], cbuf)
    pltpu.sync_copy(g_hbm.at[pl.ds(wid * _CHUNK, _CHUNK)], gbuf)

    def body(j, _):
        gv = gbuf[pl.ds(j * 16, 16)]
        obuf[pl.ds(j * 16, 16)] = plsc.load_gather(cbuf, [gv])
        return 0

    lax.fori_loop(0, _CHUNK // 16, body, 0)
    pltpu.sync_copy(obuf, out_hbm.at[pl.ds(wid * _CHUNK, _CHUNK)])


def kernel(z, codes):
    latent_dim, num_codes, channel_dim = codes.shape
    batch, channels, height, width = z.shape
    hw = height * width
    zr = z.reshape(batch, latent_dim, channel_dim, hw)

    idx4, g = pl.pallas_call(
        _tc_body,
        grid=(batch, latent_dim),
        in_specs=[
            pl.BlockSpec((1, 1, channel_dim, hw), lambda b, l: (b, l, 0, 0)),
            pl.BlockSpec((1, num_codes, channel_dim), lambda b, l: (l, 0, 0)),
        ],
        out_specs=[
            pl.BlockSpec((1, 1, 1, hw), lambda b, l: (b, l, 0, 0)),
            pl.BlockSpec((1, 1, channel_dim, hw), lambda b, l: (b, l, 0, 0)),
        ],
        out_shape=[
            jax.ShapeDtypeStruct((batch, latent_dim, 1, hw), jnp.int32),
            jax.ShapeDtypeStruct((batch, latent_dim, channel_dim, hw), jnp.int32),
        ],
    )(zr, codes)

    codes_flat = jnp.concatenate(
        [codes.reshape(latent_dim * num_codes * channel_dim),
         jnp.zeros((num_codes * channel_dim,), jnp.float32)], axis=0)

    mesh = plsc.VectorSubcoreMesh(core_axis_name="c", subcore_axis_name="s")
    hard_flat = pl.kernel(
        _sc_body,
        mesh=mesh,
        compiler_params=pltpu.CompilerParams(needs_layout_passes=False),
        out_type=jax.ShapeDtypeStruct((batch * channels * hw,), jnp.float32),
        scratch_types=[
            pltpu.VMEM((2 * num_codes * channel_dim,), jnp.float32),
            pltpu.VMEM((_CHUNK,), jnp.int32),
            pltpu.VMEM((_CHUNK,), jnp.float32),
        ],
    )(codes_flat, g.reshape(-1))

    quantized = hard_flat.reshape(batch, channels, height, width)
    idxes = (idx4.reshape(batch, latent_dim, hw)
                 .transpose(0, 2, 1)
                 .reshape(batch, height, width, latent_dim))
    return (quantized, idxes)


# speedup vs baseline: 1.0920x; 1.0920x over previous
import jax
import jax.numpy as jnp
from jax import lax
from jax.experimental import pallas as pl
from jax.experimental.pallas import tpu as pltpu
from jax.experimental.pallas import tpu_sc as plsc

_HW = 196
_K = 512
_CD = 32
_L = 12
_B = 2
_NW_USED = _B * _L
_CHUNK = _CD * _HW
_UNROLL = 8
_N16 = _CHUNK // 16


def _tc_body(z_ref, codes_ref, idx_ref, g_ref):
    i_iota = jax.lax.broadcasted_iota(jnp.int32, (_CD, _HW), 0)
    for l in range(_L):
        c = codes_ref[l]
        cn = jnp.sum(c * c, axis=1, keepdims=True)
        for b in range(_B):
            h = z_ref[b, l]
            scores = jax.lax.dot_general(
                c, h, (((1,), (0,)), ((), ())),
                preferred_element_type=jnp.float32,
                precision=jax.lax.Precision.HIGHEST)
            d2 = cn - 2.0 * scores
            m = jnp.min(d2, axis=0, keepdims=True)
            kiota = jax.lax.broadcasted_iota(jnp.int32, d2.shape, 0)
            idx = jnp.min(jnp.where(d2 == m, kiota, _K), axis=0)
            idx_ref[b, l, 0, :] = idx
            g_ref[b, l] = idx[None, :] * _CD + i_iota


def _sc_body(codes_hbm, g_hbm, out_hbm, cbuf, gbuf, obuf):
    wid = lax.axis_index("c") * 16 + lax.axis_index("s")

    @pl.when(wid < _NW_USED)
    def _():
        l = wid % _L
        pltpu.sync_copy(codes_hbm.at[pl.ds(l * _K * _CD, _K * _CD)], cbuf)
        pltpu.sync_copy(g_hbm.at[pl.ds(wid * _CHUNK, _CHUNK)], gbuf)

        def body(j, _):
            base = j * (16 * _UNROLL)
            for u in range(_UNROLL):
                off = base + u * 16
                gv = gbuf[pl.ds(off, 16)]
                obuf[pl.ds(off, 16)] = plsc.load_gather(cbuf, [gv])
            return 0

        lax.fori_loop(0, _N16 // _UNROLL, body, 0)
        pltpu.sync_copy(obuf, out_hbm.at[pl.ds(wid * _CHUNK, _CHUNK)])


def kernel(z, codes):
    latent_dim, num_codes, channel_dim = codes.shape
    batch, channels, height, width = z.shape
    hw = height * width
    zr = z.reshape(batch, latent_dim, channel_dim, hw)

    idx4, g = pl.pallas_call(
        _tc_body,
        out_shape=[
            jax.ShapeDtypeStruct((batch, latent_dim, 1, hw), jnp.int32),
            jax.ShapeDtypeStruct((batch, latent_dim, channel_dim, hw), jnp.int32),
        ],
    )(zr, codes)

    mesh = plsc.VectorSubcoreMesh(core_axis_name="c", subcore_axis_name="s")
    hard_flat = pl.kernel(
        _sc_body,
        mesh=mesh,
        compiler_params=pltpu.CompilerParams(needs_layout_passes=False),
        out_type=jax.ShapeDtypeStruct((batch * channels * hw,), jnp.float32),
        scratch_types=[
            pltpu.VMEM((num_codes * channel_dim,), jnp.float32),
            pltpu.VMEM((_CHUNK,), jnp.int32),
            pltpu.VMEM((_CHUNK,), jnp.float32),
        ],
    )(codes.reshape(-1), g.reshape(-1))

    quantized = hard_flat.reshape(batch, channels, height, width)
    idxes = (idx4.reshape(batch, latent_dim, hw)
                 .transpose(0, 2, 1)
                 .reshape(batch, height, width, latent_dim))
    return (quantized, idxes)


# speedup vs baseline: 1.6175x; 1.4812x over previous
import jax
import jax.numpy as jnp
from jax.experimental import pallas as pl

_HW = 196
_K = 512
_CD = 32
_L = 12
_B = 2


def _tc_body(z_ref, codes_ref, hard_ref, idx_ref):
    for l in range(_L):
        c = codes_ref[l]
        cn = jnp.sum(c * c, axis=1, keepdims=True)
        for b in range(_B):
            h = z_ref[b, l]
            scores = jax.lax.dot_general(
                c, h, (((1,), (0,)), ((), ())),
                preferred_element_type=jnp.float32,
                precision=jax.lax.Precision.HIGHEST)
            d2 = cn - 2.0 * scores
            m = jnp.min(d2, axis=0, keepdims=True)
            kiota = jax.lax.broadcasted_iota(jnp.int32, d2.shape, 0)
            idx = jnp.min(jnp.where(d2 == m, kiota, _K), axis=0)
            idx_ref[b, l, 0, :] = idx
            onehot = jnp.where(kiota == idx[None, :], 1.0, 0.0)
            hard_ref[b, l] = jax.lax.dot_general(
                c, onehot, (((0,), (0,)), ((), ())),
                preferred_element_type=jnp.float32,
                precision=jax.lax.Precision.DEFAULT)


def kernel(z, codes):
    latent_dim, num_codes, channel_dim = codes.shape
    batch, channels, height, width = z.shape
    hw = height * width
    zr = z.reshape(batch, latent_dim, channel_dim, hw)

    hard, idx4 = pl.pallas_call(
        _tc_body,
        out_shape=[
            jax.ShapeDtypeStruct((batch, latent_dim, channel_dim, hw), jnp.float32),
            jax.ShapeDtypeStruct((batch, latent_dim, 1, hw), jnp.int32),
        ],
    )(zr, codes)

    quantized = hard.reshape(batch, channels, height, width)
    idxes = (idx4.reshape(batch, latent_dim, hw)
                 .transpose(0, 2, 1)
                 .reshape(batch, height, width, latent_dim))
    return (quantized, idxes)
